# speedup vs baseline: 1.2746x; 1.2746x over previous
import functools

import jax
import jax.numpy as jnp
from jax import lax
from jax.experimental import pallas as pl
from jax.experimental.pallas import tpu as pltpu
from jax.experimental.pallas import tpu_sc as plsc

B = 4
S = 2048
NEMBD = 128
NW = 32
ROWS = (B * S) // NW
CHUNK = 128
NCHUNK = ROWS // CHUNK
WPW = S // ROWS
LANES = 16


def _embd_body(idx_hbm, wte_hbm, wpe_hbm, out_hbm, idx_v, rows_v, wpe_v, sem):
    c = lax.axis_index("c")
    s = lax.axis_index("s")
    wid = s * 2 + c
    base = wid * ROWS
    pos_base = lax.rem(wid, WPW) * ROWS

    pltpu.sync_copy(idx_hbm.at[pl.ds(wid * NCHUNK, NCHUNK)], idx_v)

    copies = []
    for j in range(NCHUNK):
        copies.append(
            pltpu.async_copy(
                wte_hbm.at[idx_v.at[j]],
                rows_v.at[pl.ds(j * CHUNK, CHUNK)],
                sem,
            )
        )
    pltpu.sync_copy(wpe_hbm.at[pl.ds(pos_base, ROWS)], wpe_v)
    for cp in copies:
        cp.wait()

    def add_row(r, carry):
        for j in range(NEMBD // LANES):
            sl = pl.ds(j * LANES, LANES)
            rows_v[r, sl] = rows_v[r, sl] + wpe_v[r, sl]
        return carry

    lax.fori_loop(0, ROWS, add_row, 0)

    pltpu.sync_copy(rows_v, out_hbm.at[pl.ds(base, ROWS)])


@jax.jit
def _embd(idx2d, wte, wpe):
    mesh = plsc.VectorSubcoreMesh(core_axis_name="c", subcore_axis_name="s")
    return pl.kernel(
        _embd_body,
        out_type=jax.ShapeDtypeStruct((B * S, NEMBD), jnp.float32),
        mesh=mesh,
        scratch_types=[
            pltpu.VMEM((NCHUNK, CHUNK), jnp.int32),
            pltpu.VMEM((ROWS, NEMBD), jnp.float32),
            pltpu.VMEM((ROWS, NEMBD), jnp.float32),
            pltpu.SemaphoreType.DMA,
        ],
    )(idx2d, wte, wpe)


def kernel(idx, wte, wpe):
    idx2d = idx.astype(jnp.int32).reshape(B * S // CHUNK, CHUNK)
    out = _embd(idx2d, wte, wpe)
    return out.reshape(B, S, NEMBD)


# speedup vs baseline: 1.3658x; 1.0716x over previous
import jax
import jax.numpy as jnp
from jax import lax
from jax.experimental import pallas as pl
from jax.experimental.pallas import tpu as pltpu
from jax.experimental.pallas import tpu_sc as plsc

B = 4
S = 2048
NEMBD = 128
NW = 32
POS = S // NW
LANES = 16
NVEC = NEMBD // LANES


def _embd_body(idx_hbm, wte_hbm, wpe_hbm, out_hbm,
               idx_v, rows_v, wpe_v,
               sem_i, sem_w, sem_g0, sem_g1, sem_g2, sem_g3, sem_o):
    c = lax.axis_index("c")
    s = lax.axis_index("s")
    wid = s * 2 + c
    pbase = wid * POS

    idx_copies = [
        pltpu.async_copy(idx_hbm.at[b, pl.ds(pbase, POS)], idx_v.at[b], sem_i)
        for b in range(B)
    ]
    wpe_copy = pltpu.async_copy(wpe_hbm.at[pl.ds(pbase, POS)], wpe_v, sem_w)
    for cp in idx_copies:
        cp.wait()

    gsems = [sem_g0, sem_g1, sem_g2, sem_g3]
    gathers = [
        pltpu.async_copy(
            wte_hbm.at[idx_v.at[b]], rows_v.at[pl.ds(b * POS, POS)], gsems[b]
        )
        for b in range(B)
    ]
    wpe_copy.wait()

    stores = []
    for b in range(B):
        gathers[b].wait()

        def add_pos(p, carry, b=b):
            r = b * POS + p
            for j in range(NVEC):
                sl = pl.ds(j * LANES, LANES)
                rows_v[r, sl] = rows_v[r, sl] + wpe_v[p, sl]
            return carry

        lax.fori_loop(0, POS, add_pos, 0)
        stores.append(
            pltpu.async_copy(
                rows_v.at[pl.ds(b * POS, POS)],
                out_hbm.at[pl.ds(b * S + pbase, POS)],
                sem_o,
            )
        )
    for st in stores:
        st.wait()


@jax.jit
def _embd(idx, wte, wpe):
    mesh = plsc.VectorSubcoreMesh(core_axis_name="c", subcore_axis_name="s")
    return pl.kernel(
        _embd_body,
        out_type=jax.ShapeDtypeStruct((B * S, NEMBD), jnp.float32),
        mesh=mesh,
        scratch_types=[
            pltpu.VMEM((B, POS), jnp.int32),
            pltpu.VMEM((B * POS, NEMBD), jnp.float32),
            pltpu.VMEM((POS, NEMBD), jnp.float32),
        ] + [pltpu.SemaphoreType.DMA] * 7,
    )(idx, wte, wpe)


def kernel(idx, wte, wpe):
    out = _embd(idx.astype(jnp.int32), wte, wpe)
    return out.reshape(B, S, NEMBD)


# speedup vs baseline: 1.3960x; 1.0221x over previous
import jax
import jax.numpy as jnp
from jax import lax
from jax.experimental import pallas as pl
from jax.experimental.pallas import tpu as pltpu
from jax.experimental.pallas import tpu_sc as plsc

B = 4
S = 2048
NEMBD = 128
NW = 32
POS = S // NW
LANES = 16
NVEC = NEMBD // LANES


def _embd_body(idx_hbm, wte_hbm, wpe_hbm, out_hbm,
               idx_v, rows_v, wpe_v,
               sem_i, sem_w, sem_g0, sem_g1, sem_o):
    c = lax.axis_index("c")
    s = lax.axis_index("s")
    wid = s * 2 + c
    pbase = wid * POS

    idx_copies = [
        pltpu.async_copy(
            idx_hbm.at[b, pl.ds(pbase, POS)],
            idx_v.at[b // 2, pl.ds((b % 2) * POS, POS)],
            sem_i,
        )
        for b in range(B)
    ]
    wpe_copy = pltpu.async_copy(wpe_hbm.at[pl.ds(pbase, POS)], wpe_v, sem_w)
    for cp in idx_copies:
        cp.wait()

    gathers = [
        pltpu.async_copy(
            wte_hbm.at[idx_v.at[h]],
            rows_v.at[pl.ds(h * 2 * POS, 2 * POS)],
            sem,
        )
        for h, sem in ((0, sem_g0), (1, sem_g1))
    ]
    wpe_copy.wait()

    stores = []
    for h in range(2):
        gathers[h].wait()

        def add_pos(p, carry, h=h):
            for j in range(NVEC):
                sl = pl.ds(j * LANES, LANES)
                w = wpe_v[p, sl]
                for bb in range(2):
                    r = (h * 2 + bb) * POS + p
                    rows_v[r, sl] = rows_v[r, sl] + w
            return carry

        lax.fori_loop(0, POS, add_pos, 0)
        for bb in range(2):
            b = h * 2 + bb
            stores.append(
                pltpu.async_copy(
                    rows_v.at[pl.ds(b * POS, POS)],
                    out_hbm.at[pl.ds(b * S + pbase, POS)],
                    sem_o,
                )
            )
    for st in stores:
        st.wait()


@jax.jit
def _embd(idx, wte, wpe):
    mesh = plsc.VectorSubcoreMesh(core_axis_name="c", subcore_axis_name="s")
    return pl.kernel(
        _embd_body,
        out_type=jax.ShapeDtypeStruct((B * S, NEMBD), jnp.float32),
        mesh=mesh,
        scratch_types=[
            pltpu.VMEM((2, 2 * POS), jnp.int32),
            pltpu.VMEM((B * POS, NEMBD), jnp.float32),
            pltpu.VMEM((POS, NEMBD), jnp.float32),
        ] + [pltpu.SemaphoreType.DMA] * 5,
    )(idx, wte, wpe)


def kernel(idx, wte, wpe):
    out = _embd(idx.astype(jnp.int32), wte, wpe)
    return out.reshape(B, S, NEMBD)
